# baseline (speedup 1.0000x reference)
import jax
import jax.numpy as jnp
from jax import lax
from jax.experimental import pallas as pl
from jax.experimental.pallas import tpu as pltpu
from jax.experimental.pallas import tpu_sc as plsc

N = 10000
E = 320000
C = 128
NU = 1024
EPS = 1e-5

NC = 2
NS = 16
NW = NC * NS
EPW = E // NW
KCH = 50
NCHUNK = EPW // KCH
NP = 10240
RPT = NP // NS
CW = 16


GSZ = 8
NGRP = NCHUNK // GSZ
RING = 4
PRE = 2


def _seg_body(h_hbm, src_hbm, dst_hbm, zrow_hbm, agg_out,
              si, di, rows_v, agg_sh, gsem, ssem, isem):
    c = lax.axis_index("c")
    s = lax.axis_index("s")
    wid = s * NC + c

    base = pl.multiple_of(s * RPT, 8)
    pltpu.sync_copy(zrow_hbm, agg_sh.at[pl.ds(base, RPT)])

    pltpu.sync_copy(src_hbm.at[wid, pl.ds(0, GSZ)], si.at[0])
    pltpu.sync_copy(dst_hbm.at[wid, pl.ds(0, GSZ)], di.at[0])
    pltpu.async_copy(src_hbm.at[wid, pl.ds(GSZ, GSZ)], si.at[1], isem.at[1])
    pltpu.async_copy(dst_hbm.at[wid, pl.ds(GSZ, GSZ)], di.at[1], isem.at[1])
    plsc.subcore_barrier()
    for j in range(PRE):
        pltpu.async_copy(h_hbm.at[si.at[0, j]], rows_v.at[j], gsem.at[j])

    def group(g, carry):
        p = g % 2
        for b in range(GSZ):
            q = (g * GSZ + b) % RING
            pltpu.make_async_copy(h_hbm.at[si.at[p, b]], rows_v.at[q],
                                  gsem.at[q]).wait()

            nb = (b + PRE) % RING
            bn = b + PRE
            if b >= PRE:
                pltpu.make_async_copy(rows_v.at[nb], agg_sh.at[di.at[p, b]],
                                      ssem.at[nb]).wait()
            else:
                @pl.when(g >= 1)
                def _():
                    pltpu.make_async_copy(rows_v.at[nb],
                                          agg_sh.at[di.at[p, b]],
                                          ssem.at[nb]).wait()
            if b == PRE:
                @pl.when((g >= 1) & (g + 1 < NGRP))
                def _():
                    nxt = pl.multiple_of((g + 1) * GSZ, 8)
                    pltpu.async_copy(src_hbm.at[wid, pl.ds(nxt, GSZ)],
                                     si.at[1 - p], isem.at[1 - p])
                    pltpu.async_copy(dst_hbm.at[wid, pl.ds(nxt, GSZ)],
                                     di.at[1 - p], isem.at[1 - p])
            if bn < GSZ:
                pltpu.async_copy(h_hbm.at[si.at[p, bn]], rows_v.at[nb],
                                 gsem.at[nb])
            elif bn - GSZ == 0:
                @pl.when(g + 1 < NGRP)
                def _():
                    pltpu.make_async_copy(src_hbm.at[wid, pl.ds(0, GSZ)],
                                          si.at[1 - p], isem.at[1 - p]).wait()
                    pltpu.make_async_copy(dst_hbm.at[wid, pl.ds(0, GSZ)],
                                          di.at[1 - p], isem.at[1 - p]).wait()
                    pltpu.async_copy(h_hbm.at[si.at[1 - p, 0]],
                                     rows_v.at[nb], gsem.at[nb])
            else:
                @pl.when(g + 1 < NGRP)
                def _():
                    pltpu.async_copy(h_hbm.at[si.at[1 - p, bn - GSZ]],
                                     rows_v.at[nb], gsem.at[nb])

            pltpu.async_copy(rows_v.at[q], agg_sh.at[di.at[p, b]],
                             ssem.at[q], add=True)
        return carry

    lax.fori_loop(0, NGRP, group, 0)

    for j in range(NCHUNK - PRE, NCHUNK):
        pltpu.make_async_copy(rows_v.at[j % RING], agg_sh.at[di.at[1, 0]],
                              ssem.at[j % RING]).wait()
    plsc.subcore_barrier()

    pltpu.sync_copy(agg_sh.at[pl.ds(base, RPT)],
                    agg_out.at[c, pl.ds(base, RPT)])


_seg_kernel = pl.kernel(
    _seg_body,
    out_type=jax.ShapeDtypeStruct((NC, NP, C), jnp.float32),
    mesh=plsc.VectorSubcoreMesh(core_axis_name="c", subcore_axis_name="s"),
    scratch_types=(
        pltpu.VMEM((2, GSZ, KCH), jnp.int32),
        pltpu.VMEM((2, GSZ, KCH), jnp.int32),
        pltpu.VMEM((RING, KCH, C), jnp.float32),
        pltpu.VMEM_SHARED((NP, C), jnp.float32),
        pltpu.SemaphoreType.DMA((RING,)),
        pltpu.SemaphoreType.DMA((RING,)),
        pltpu.SemaphoreType.DMA((2,)),
    ),
)


def _cnt_body(dst_hbm, zcnt_hbm, ones_hbm, cnt_out,
              dst_v, ones_v, cnt_sh):
    c = lax.axis_index("c")
    s = lax.axis_index("s")
    wid = s * NC + c

    pltpu.sync_copy(dst_hbm.at[wid], dst_v)
    base = pl.multiple_of(s * RPT, 8)
    pltpu.sync_copy(zcnt_hbm, cnt_sh.at[pl.ds(base, RPT)])
    pltpu.sync_copy(ones_hbm, ones_v)
    plsc.subcore_barrier()

    def chunk(j, carry):
        pltpu.sync_copy(ones_v, cnt_sh.at[dst_v.at[j]], add=True)
        return carry

    lax.fori_loop(0, NCHUNK, chunk, 0)
    plsc.subcore_barrier()

    pltpu.sync_copy(cnt_sh.at[pl.ds(base, RPT)],
                    cnt_out.at[c, pl.ds(base, RPT)])


_cnt_kernel = pl.kernel(
    _cnt_body,
    out_type=jax.ShapeDtypeStruct((NC, NP), jnp.float32),
    mesh=plsc.VectorSubcoreMesh(core_axis_name="c", subcore_axis_name="s"),
    scratch_types=(
        pltpu.VMEM((NCHUNK, KCH), jnp.int32),
        pltpu.VMEM((KCH,), jnp.float32),
        pltpu.VMEM_SHARED((NP,), jnp.float32),
    ),
)


_UPW = NU // NW


def _gather_body(h_hbm, uidx_hbm, out_hbm, idx_v, rows_v, sem):
    c = lax.axis_index("c")
    s = lax.axis_index("s")
    wid = s * NC + c
    pltpu.sync_copy(uidx_hbm.at[wid], idx_v)
    pltpu.async_copy(h_hbm.at[idx_v], rows_v, sem).wait()
    pltpu.sync_copy(rows_v, out_hbm.at[pl.ds(pl.multiple_of(wid * _UPW, 8), _UPW)])


_gather_kernel = pl.kernel(
    _gather_body,
    out_type=jax.ShapeDtypeStruct((NU, C), jnp.float32),
    mesh=plsc.VectorSubcoreMesh(core_axis_name="c", subcore_axis_name="s"),
    scratch_types=(
        pltpu.VMEM((_UPW,), jnp.int32),
        pltpu.VMEM((_UPW, C), jnp.float32),
        pltpu.SemaphoreType.DMA,
    ),
)


def _bn_relu(z, g, b):
    m = jnp.mean(z, axis=0, keepdims=True)
    d = z - m
    v = jnp.mean(d * d, axis=0, keepdims=True)
    return jnp.maximum(d * lax.rsqrt(v + EPS) * g + b, 0.0)


def _tc_layer1_body(aggp, cntp, h, wlT, bl, wrT, g, b, out, cnt_out):
    cnt = jnp.maximum(cntp[0][:N] + cntp[1][:N], 1.0)[:, None]
    cnt_out[...] = cnt
    mean = (aggp[0][:N] + aggp[1][:N]) / cnt
    z = (jnp.dot(mean, wlT[...], preferred_element_type=jnp.float32)
         + bl[...]
         + jnp.dot(h[...], wrT[...], preferred_element_type=jnp.float32))
    out[...] = _bn_relu(z, g[...], b[...])


_tc_layer1 = pl.pallas_call(
    _tc_layer1_body,
    out_shape=(
        jax.ShapeDtypeStruct((N, C), jnp.float32),
        jax.ShapeDtypeStruct((N, 1), jnp.float32),
    ),
)


def _tc_layer_body(aggp, cnt, h, wlT, bl, wrT, g, b, out):
    mean = (aggp[0][:N] + aggp[1][:N]) / cnt[...]
    z = (jnp.dot(mean, wlT[...], preferred_element_type=jnp.float32)
         + bl[...]
         + jnp.dot(h[...], wrT[...], preferred_element_type=jnp.float32))
    out[...] = _bn_relu(z, g[...], b[...])


_tc_layer = pl.pallas_call(
    _tc_layer_body,
    out_shape=jax.ShapeDtypeStruct((N, C), jnp.float32),
)


def _tc_final_body(h, mask, w1T, b1, g2, bt2, w2T, b2, atom_out):
    a = _bn_relu(
        jnp.dot(h[...], w1T[...], preferred_element_type=jnp.float32) + b1[...],
        g2[...], bt2[...])
    atom_out[...] = (
        jnp.dot(a, w2T[...], preferred_element_type=jnp.float32) + b2[...]
    ) * mask[...]


_tc_final = pl.pallas_call(
    _tc_final_body,
    out_shape=jax.ShapeDtypeStruct((N, C), jnp.float32),
)


def kernel(x, edge_index, atom_mask, u_index, Wl, bl, Wr, gamma, beta,
           W1, b1, g2, bt2, W2, b2):
    src3 = edge_index[0].reshape(NW, NCHUNK, KCH)
    dst3 = edge_index[1].reshape(NW, NCHUNK, KCH)
    zrow = jnp.zeros((RPT, C), jnp.float32)
    zcnt = jnp.zeros((RPT,), jnp.float32)
    ones_k = jnp.ones((KCH,), jnp.float32)
    u3 = u_index.reshape(NW, _UPW)

    cntp = _cnt_kernel(dst3, zcnt, ones_k)
    aggp = _seg_kernel(x, src3, dst3, zrow)
    h1, cnt = _tc_layer1(aggp, cntp, x,
                         Wl[0].T, bl[0], Wr[0].T, gamma[0], beta[0])
    aggp2 = _seg_kernel(h1, src3, dst3, zrow)
    h2 = _tc_layer(aggp2, cnt, h1, Wl[1].T, bl[1], Wr[1].T, gamma[1], beta[1])
    aggp3 = _seg_kernel(h2, src3, dst3, zrow)
    h3 = _tc_layer(aggp3, cnt, h2, Wl[2].T, bl[2], Wr[2].T, gamma[2], beta[2])

    state = _gather_kernel(h3, u3)
    atom = _tc_final(h3, atom_mask, W1.T, b1, g2, bt2, W2.T, b2)
    return (h3, atom, state)


# speedup vs baseline: 1.0304x; 1.0304x over previous
import jax
import jax.numpy as jnp
from jax import lax
from jax.experimental import pallas as pl
from jax.experimental.pallas import tpu as pltpu
from jax.experimental.pallas import tpu_sc as plsc

N = 10000
E = 320000
C = 128
NU = 1024
EPS = 1e-5

NC = 2
NS = 16
NW = NC * NS
EPW = E // NW
KCH = 125
NCHUNK = EPW // KCH
NP = 10240
RPT = NP // NS
CW = 16


GSZ = 8
NGRP = NCHUNK // GSZ
RING = 2
PRE = 1
LAG = RING - PRE


def _seg_body(h_hbm, src_hbm, dst_hbm, zrow_hbm, agg_out,
              si, di, rows_v, agg_sh, gsem, ssem, isem):
    c = lax.axis_index("c")
    s = lax.axis_index("s")
    wid = s * NC + c

    base = pl.multiple_of(s * RPT, 8)
    pltpu.sync_copy(zrow_hbm, agg_sh.at[pl.ds(base, RPT)])

    pltpu.sync_copy(src_hbm.at[wid, pl.ds(0, GSZ)], si.at[0])
    pltpu.sync_copy(dst_hbm.at[wid, pl.ds(0, GSZ)], di.at[0])
    pltpu.async_copy(src_hbm.at[wid, pl.ds(GSZ, GSZ)], si.at[1], isem.at[1])
    pltpu.async_copy(dst_hbm.at[wid, pl.ds(GSZ, GSZ)], di.at[1], isem.at[1])
    plsc.subcore_barrier()
    for j in range(PRE):
        pltpu.async_copy(h_hbm.at[si.at[0, j]], rows_v.at[j], gsem.at[j])

    def group(g, carry):
        p = g % 2
        for b in range(GSZ):
            q = (g * GSZ + b) % RING
            pltpu.make_async_copy(h_hbm.at[si.at[p, b]], rows_v.at[q],
                                  gsem.at[q]).wait()

            nb = (b + PRE) % RING
            bn = b + PRE
            if b >= LAG:
                pltpu.make_async_copy(rows_v.at[nb], agg_sh.at[di.at[p, b]],
                                      ssem.at[nb]).wait()
            else:
                @pl.when(g >= 1)
                def _():
                    pltpu.make_async_copy(rows_v.at[nb],
                                          agg_sh.at[di.at[p, b]],
                                          ssem.at[nb]).wait()
            if b == PRE:
                @pl.when((g >= 1) & (g + 1 < NGRP))
                def _():
                    nxt = pl.multiple_of((g + 1) * GSZ, 8)
                    pltpu.async_copy(src_hbm.at[wid, pl.ds(nxt, GSZ)],
                                     si.at[1 - p], isem.at[1 - p])
                    pltpu.async_copy(dst_hbm.at[wid, pl.ds(nxt, GSZ)],
                                     di.at[1 - p], isem.at[1 - p])
            if bn < GSZ:
                pltpu.async_copy(h_hbm.at[si.at[p, bn]], rows_v.at[nb],
                                 gsem.at[nb])
            elif bn - GSZ == 0:
                @pl.when(g + 1 < NGRP)
                def _():
                    pltpu.make_async_copy(src_hbm.at[wid, pl.ds(0, GSZ)],
                                          si.at[1 - p], isem.at[1 - p]).wait()
                    pltpu.make_async_copy(dst_hbm.at[wid, pl.ds(0, GSZ)],
                                          di.at[1 - p], isem.at[1 - p]).wait()
                    pltpu.async_copy(h_hbm.at[si.at[1 - p, 0]],
                                     rows_v.at[nb], gsem.at[nb])
            else:
                @pl.when(g + 1 < NGRP)
                def _():
                    pltpu.async_copy(h_hbm.at[si.at[1 - p, bn - GSZ]],
                                     rows_v.at[nb], gsem.at[nb])

            pltpu.async_copy(rows_v.at[q], agg_sh.at[di.at[p, b]],
                             ssem.at[q], add=True)
        return carry

    lax.fori_loop(0, NGRP, group, 0)

    for j in range(NCHUNK - LAG, NCHUNK):
        pltpu.make_async_copy(rows_v.at[j % RING], agg_sh.at[di.at[1, 0]],
                              ssem.at[j % RING]).wait()
    plsc.subcore_barrier()

    pltpu.sync_copy(agg_sh.at[pl.ds(base, RPT)],
                    agg_out.at[c, pl.ds(base, RPT)])


_seg_kernel = pl.kernel(
    _seg_body,
    out_type=jax.ShapeDtypeStruct((NC, NP, C), jnp.float32),
    mesh=plsc.VectorSubcoreMesh(core_axis_name="c", subcore_axis_name="s"),
    scratch_types=(
        pltpu.VMEM((2, GSZ, KCH), jnp.int32),
        pltpu.VMEM((2, GSZ, KCH), jnp.int32),
        pltpu.VMEM((RING, KCH, C), jnp.float32),
        pltpu.VMEM_SHARED((NP, C), jnp.float32),
        pltpu.SemaphoreType.DMA((RING,)),
        pltpu.SemaphoreType.DMA((RING,)),
        pltpu.SemaphoreType.DMA((2,)),
    ),
)


def _cnt_body(dst_hbm, zcnt_hbm, ones_hbm, cnt_out,
              dst_v, ones_v, cnt_sh):
    c = lax.axis_index("c")
    s = lax.axis_index("s")
    wid = s * NC + c

    pltpu.sync_copy(dst_hbm.at[wid], dst_v)
    base = pl.multiple_of(s * RPT, 8)
    pltpu.sync_copy(zcnt_hbm, cnt_sh.at[pl.ds(base, RPT)])
    pltpu.sync_copy(ones_hbm, ones_v)
    plsc.subcore_barrier()

    def chunk(j, carry):
        pltpu.sync_copy(ones_v, cnt_sh.at[dst_v.at[j]], add=True)
        return carry

    lax.fori_loop(0, NCHUNK, chunk, 0)
    plsc.subcore_barrier()

    pltpu.sync_copy(cnt_sh.at[pl.ds(base, RPT)],
                    cnt_out.at[c, pl.ds(base, RPT)])


_cnt_kernel = pl.kernel(
    _cnt_body,
    out_type=jax.ShapeDtypeStruct((NC, NP), jnp.float32),
    mesh=plsc.VectorSubcoreMesh(core_axis_name="c", subcore_axis_name="s"),
    scratch_types=(
        pltpu.VMEM((NCHUNK, KCH), jnp.int32),
        pltpu.VMEM((KCH,), jnp.float32),
        pltpu.VMEM_SHARED((NP,), jnp.float32),
    ),
)


_UPW = NU // NW


def _gather_body(h_hbm, uidx_hbm, out_hbm, idx_v, rows_v, sem):
    c = lax.axis_index("c")
    s = lax.axis_index("s")
    wid = s * NC + c
    pltpu.sync_copy(uidx_hbm.at[wid], idx_v)
    pltpu.async_copy(h_hbm.at[idx_v], rows_v, sem).wait()
    pltpu.sync_copy(rows_v, out_hbm.at[pl.ds(pl.multiple_of(wid * _UPW, 8), _UPW)])


_gather_kernel = pl.kernel(
    _gather_body,
    out_type=jax.ShapeDtypeStruct((NU, C), jnp.float32),
    mesh=plsc.VectorSubcoreMesh(core_axis_name="c", subcore_axis_name="s"),
    scratch_types=(
        pltpu.VMEM((_UPW,), jnp.int32),
        pltpu.VMEM((_UPW, C), jnp.float32),
        pltpu.SemaphoreType.DMA,
    ),
)


def _bn_relu(z, g, b):
    m = jnp.mean(z, axis=0, keepdims=True)
    d = z - m
    v = jnp.mean(d * d, axis=0, keepdims=True)
    return jnp.maximum(d * lax.rsqrt(v + EPS) * g + b, 0.0)


def _tc_layer1_body(aggp, cntp, h, wlT, bl, wrT, g, b, out, cnt_out):
    cnt = jnp.maximum(cntp[0][:N] + cntp[1][:N], 1.0)[:, None]
    cnt_out[...] = cnt
    mean = (aggp[0][:N] + aggp[1][:N]) / cnt
    z = (jnp.dot(mean, wlT[...], preferred_element_type=jnp.float32)
         + bl[...]
         + jnp.dot(h[...], wrT[...], preferred_element_type=jnp.float32))
    out[...] = _bn_relu(z, g[...], b[...])


_tc_layer1 = pl.pallas_call(
    _tc_layer1_body,
    out_shape=(
        jax.ShapeDtypeStruct((N, C), jnp.float32),
        jax.ShapeDtypeStruct((N, 1), jnp.float32),
    ),
)


def _tc_layer_body(aggp, cnt, h, wlT, bl, wrT, g, b, out):
    mean = (aggp[0][:N] + aggp[1][:N]) / cnt[...]
    z = (jnp.dot(mean, wlT[...], preferred_element_type=jnp.float32)
         + bl[...]
         + jnp.dot(h[...], wrT[...], preferred_element_type=jnp.float32))
    out[...] = _bn_relu(z, g[...], b[...])


_tc_layer = pl.pallas_call(
    _tc_layer_body,
    out_shape=jax.ShapeDtypeStruct((N, C), jnp.float32),
)


def _tc_final_body(h, mask, w1T, b1, g2, bt2, w2T, b2, atom_out):
    a = _bn_relu(
        jnp.dot(h[...], w1T[...], preferred_element_type=jnp.float32) + b1[...],
        g2[...], bt2[...])
    atom_out[...] = (
        jnp.dot(a, w2T[...], preferred_element_type=jnp.float32) + b2[...]
    ) * mask[...]


_tc_final = pl.pallas_call(
    _tc_final_body,
    out_shape=jax.ShapeDtypeStruct((N, C), jnp.float32),
)


def kernel(x, edge_index, atom_mask, u_index, Wl, bl, Wr, gamma, beta,
           W1, b1, g2, bt2, W2, b2):
    src3 = edge_index[0].reshape(NW, NCHUNK, KCH)
    dst3 = edge_index[1].reshape(NW, NCHUNK, KCH)
    zrow = jnp.zeros((RPT, C), jnp.float32)
    zcnt = jnp.zeros((RPT,), jnp.float32)
    ones_k = jnp.ones((KCH,), jnp.float32)
    u3 = u_index.reshape(NW, _UPW)

    cntp = _cnt_kernel(dst3, zcnt, ones_k)
    aggp = _seg_kernel(x, src3, dst3, zrow)
    h1, cnt = _tc_layer1(aggp, cntp, x,
                         Wl[0].T, bl[0], Wr[0].T, gamma[0], beta[0])
    aggp2 = _seg_kernel(h1, src3, dst3, zrow)
    h2 = _tc_layer(aggp2, cnt, h1, Wl[1].T, bl[1], Wr[1].T, gamma[1], beta[1])
    aggp3 = _seg_kernel(h2, src3, dst3, zrow)
    h3 = _tc_layer(aggp3, cnt, h2, Wl[2].T, bl[2], Wr[2].T, gamma[2], beta[2])

    state = _gather_kernel(h3, u3)
    atom = _tc_final(h3, atom_mask, W1.T, b1, g2, bt2, W2.T, b2)
    return (h3, atom, state)


# speedup vs baseline: 1.0646x; 1.0331x over previous
import functools

import jax
import jax.numpy as jnp
from jax import lax
from jax.experimental import pallas as pl
from jax.experimental.pallas import tpu as pltpu
from jax.experimental.pallas import tpu_sc as plsc

N = 10000
E = 320000
C = 128
NU = 1024
EPS = 1e-5

NC = 2
NS = 16
NW = NC * NS
EPW = E // NW
KCH = 125
NCHUNK = EPW // KCH
NP = 10240
RPT = NP // NS
CW = 16


GSZ = 8
NGRP = NCHUNK // GSZ
RING = 2
PRE = 1
LAG = RING - PRE


def _seg_body(with_cnt, *refs):
    if with_cnt:
        (h_hbm, src_hbm, dst_hbm, zrow_hbm, zcnt_hbm, ones_hbm,
         agg_out, cnt_out,
         si, di, rows_v, ones_v, agg_sh, cnt_sh, gsem, ssem, isem) = refs
    else:
        (h_hbm, src_hbm, dst_hbm, zrow_hbm,
         agg_out,
         si, di, rows_v, agg_sh, gsem, ssem, isem) = refs
    c = lax.axis_index("c")
    s = lax.axis_index("s")
    wid = s * NC + c

    base = pl.multiple_of(s * RPT, 8)
    pltpu.sync_copy(zrow_hbm, agg_sh.at[pl.ds(base, RPT)])
    if with_cnt:
        pltpu.sync_copy(zcnt_hbm, cnt_sh.at[pl.ds(base, RPT)])
        pltpu.sync_copy(ones_hbm, ones_v)

    pltpu.sync_copy(src_hbm.at[wid, pl.ds(0, GSZ)], si.at[0])
    pltpu.sync_copy(dst_hbm.at[wid, pl.ds(0, GSZ)], di.at[0])
    pltpu.async_copy(src_hbm.at[wid, pl.ds(GSZ, GSZ)], si.at[1], isem.at[1])
    pltpu.async_copy(dst_hbm.at[wid, pl.ds(GSZ, GSZ)], di.at[1], isem.at[1])
    plsc.subcore_barrier()
    for j in range(PRE):
        pltpu.async_copy(h_hbm.at[si.at[0, j]], rows_v.at[j], gsem.at[j])

    def group(g, carry):
        p = g % 2
        for b in range(GSZ):
            q = (g * GSZ + b) % RING
            pltpu.make_async_copy(h_hbm.at[si.at[p, b]], rows_v.at[q],
                                  gsem.at[q]).wait()

            nb = (b + PRE) % RING
            bn = b + PRE
            if b >= LAG:
                pltpu.make_async_copy(rows_v.at[nb], agg_sh.at[di.at[p, b]],
                                      ssem.at[nb]).wait()
            else:
                @pl.when(g >= 1)
                def _():
                    pltpu.make_async_copy(rows_v.at[nb],
                                          agg_sh.at[di.at[p, b]],
                                          ssem.at[nb]).wait()
            if b == PRE:
                @pl.when((g >= 1) & (g + 1 < NGRP))
                def _():
                    nxt = pl.multiple_of((g + 1) * GSZ, 8)
                    pltpu.async_copy(src_hbm.at[wid, pl.ds(nxt, GSZ)],
                                     si.at[1 - p], isem.at[1 - p])
                    pltpu.async_copy(dst_hbm.at[wid, pl.ds(nxt, GSZ)],
                                     di.at[1 - p], isem.at[1 - p])
            if bn < GSZ:
                pltpu.async_copy(h_hbm.at[si.at[p, bn]], rows_v.at[nb],
                                 gsem.at[nb])
            elif bn - GSZ == 0:
                @pl.when(g + 1 < NGRP)
                def _():
                    pltpu.make_async_copy(src_hbm.at[wid, pl.ds(0, GSZ)],
                                          si.at[1 - p], isem.at[1 - p]).wait()
                    pltpu.make_async_copy(dst_hbm.at[wid, pl.ds(0, GSZ)],
                                          di.at[1 - p], isem.at[1 - p]).wait()
                    pltpu.async_copy(h_hbm.at[si.at[1 - p, 0]],
                                     rows_v.at[nb], gsem.at[nb])
            else:
                @pl.when(g + 1 < NGRP)
                def _():
                    pltpu.async_copy(h_hbm.at[si.at[1 - p, bn - GSZ]],
                                     rows_v.at[nb], gsem.at[nb])

            pltpu.async_copy(rows_v.at[q], agg_sh.at[di.at[p, b]],
                             ssem.at[q], add=True)
            if with_cnt:
                pltpu.sync_copy(ones_v, cnt_sh.at[di.at[p, b]], add=True)
        return carry

    lax.fori_loop(0, NGRP, group, 0)

    for j in range(NCHUNK - LAG, NCHUNK):
        pltpu.make_async_copy(rows_v.at[j % RING], agg_sh.at[di.at[1, 0]],
                              ssem.at[j % RING]).wait()
    plsc.subcore_barrier()

    pltpu.sync_copy(agg_sh.at[pl.ds(base, RPT)],
                    agg_out.at[c, pl.ds(base, RPT)])
    if with_cnt:
        pltpu.sync_copy(cnt_sh.at[pl.ds(base, RPT)],
                        cnt_out.at[c, pl.ds(base, RPT)])


def _make_seg_kernel(with_cnt):
    out_type = [jax.ShapeDtypeStruct((NC, NP, C), jnp.float32)]
    scratch = [
        pltpu.VMEM((2, GSZ, KCH), jnp.int32),
        pltpu.VMEM((2, GSZ, KCH), jnp.int32),
        pltpu.VMEM((RING, KCH, C), jnp.float32),
    ]
    if with_cnt:
        out_type.append(jax.ShapeDtypeStruct((NC, NP), jnp.float32))
        scratch.append(pltpu.VMEM((KCH,), jnp.float32))
    scratch.append(pltpu.VMEM_SHARED((NP, C), jnp.float32))
    if with_cnt:
        scratch.append(pltpu.VMEM_SHARED((NP,), jnp.float32))
    scratch += [
        pltpu.SemaphoreType.DMA((RING,)),
        pltpu.SemaphoreType.DMA((RING,)),
        pltpu.SemaphoreType.DMA((2,)),
    ]
    return pl.kernel(
        functools.partial(_seg_body, with_cnt),
        out_type=tuple(out_type),
        mesh=plsc.VectorSubcoreMesh(core_axis_name="c", subcore_axis_name="s"),
        scratch_types=tuple(scratch),
    )


_seg_cnt_kernel = _make_seg_kernel(True)
_seg_kernel = _make_seg_kernel(False)


_UPW = NU // NW


def _gather_body(h_hbm, uidx_hbm, out_hbm, idx_v, rows_v, sem):
    c = lax.axis_index("c")
    s = lax.axis_index("s")
    wid = s * NC + c
    pltpu.sync_copy(uidx_hbm.at[wid], idx_v)
    pltpu.async_copy(h_hbm.at[idx_v], rows_v, sem).wait()
    pltpu.sync_copy(rows_v, out_hbm.at[pl.ds(pl.multiple_of(wid * _UPW, 8), _UPW)])


_gather_kernel = pl.kernel(
    _gather_body,
    out_type=jax.ShapeDtypeStruct((NU, C), jnp.float32),
    mesh=plsc.VectorSubcoreMesh(core_axis_name="c", subcore_axis_name="s"),
    scratch_types=(
        pltpu.VMEM((_UPW,), jnp.int32),
        pltpu.VMEM((_UPW, C), jnp.float32),
        pltpu.SemaphoreType.DMA,
    ),
)


def _bn_relu(z, g, b):
    m = jnp.mean(z, axis=0, keepdims=True)
    d = z - m
    v = jnp.mean(d * d, axis=0, keepdims=True)
    return jnp.maximum(d * lax.rsqrt(v + EPS) * g + b, 0.0)


def _tc_layer1_body(aggp, cntp, h, wlT, bl, wrT, g, b, out, cnt_out):
    cnt = jnp.maximum(cntp[0][:N] + cntp[1][:N], 1.0)[:, None]
    cnt_out[...] = cnt
    mean = (aggp[0][:N] + aggp[1][:N]) / cnt
    z = (jnp.dot(mean, wlT[...], preferred_element_type=jnp.float32)
         + bl[...]
         + jnp.dot(h[...], wrT[...], preferred_element_type=jnp.float32))
    out[...] = _bn_relu(z, g[...], b[...])


_tc_layer1 = pl.pallas_call(
    _tc_layer1_body,
    out_shape=(
        jax.ShapeDtypeStruct((N, C), jnp.float32),
        jax.ShapeDtypeStruct((N, 1), jnp.float32),
    ),
)


def _tc_layer_body(aggp, cnt, h, wlT, bl, wrT, g, b, out):
    mean = (aggp[0][:N] + aggp[1][:N]) / cnt[...]
    z = (jnp.dot(mean, wlT[...], preferred_element_type=jnp.float32)
         + bl[...]
         + jnp.dot(h[...], wrT[...], preferred_element_type=jnp.float32))
    out[...] = _bn_relu(z, g[...], b[...])


_tc_layer = pl.pallas_call(
    _tc_layer_body,
    out_shape=jax.ShapeDtypeStruct((N, C), jnp.float32),
)


def _tc_layer3_body(aggp, cnt, h, wlT, bl, wrT, g, b,
                    mask, w1T, b1, g2, bt2, w2T, b2, h_out, atom_out):
    mean = (aggp[0][:N] + aggp[1][:N]) / cnt[...]
    z = (jnp.dot(mean, wlT[...], preferred_element_type=jnp.float32)
         + bl[...]
         + jnp.dot(h[...], wrT[...], preferred_element_type=jnp.float32))
    h3 = _bn_relu(z, g[...], b[...])
    h_out[...] = h3
    a = _bn_relu(
        jnp.dot(h3, w1T[...], preferred_element_type=jnp.float32) + b1[...],
        g2[...], bt2[...])
    atom_out[...] = (
        jnp.dot(a, w2T[...], preferred_element_type=jnp.float32) + b2[...]
    ) * mask[...]


_tc_layer3 = pl.pallas_call(
    _tc_layer3_body,
    out_shape=(
        jax.ShapeDtypeStruct((N, C), jnp.float32),
        jax.ShapeDtypeStruct((N, C), jnp.float32),
    ),
)


def kernel(x, edge_index, atom_mask, u_index, Wl, bl, Wr, gamma, beta,
           W1, b1, g2, bt2, W2, b2):
    src3 = edge_index[0].reshape(NW, NCHUNK, KCH)
    dst3 = edge_index[1].reshape(NW, NCHUNK, KCH)
    zrow = jnp.zeros((RPT, C), jnp.float32)
    zcnt = jnp.zeros((RPT,), jnp.float32)
    ones_k = jnp.ones((KCH,), jnp.float32)
    u3 = u_index.reshape(NW, _UPW)

    aggp, cntp = _seg_cnt_kernel(x, src3, dst3, zrow, zcnt, ones_k)
    h1, cnt = _tc_layer1(aggp, cntp, x,
                         Wl[0].T, bl[0], Wr[0].T, gamma[0], beta[0])
    aggp2, = _seg_kernel(h1, src3, dst3, zrow)
    h2 = _tc_layer(aggp2, cnt, h1, Wl[1].T, bl[1], Wr[1].T, gamma[1], beta[1])
    aggp3, = _seg_kernel(h2, src3, dst3, zrow)
    h3, atom = _tc_layer3(aggp3, cnt, h2, Wl[2].T, bl[2], Wr[2].T,
                          gamma[2], beta[2],
                          atom_mask, W1.T, b1, g2, bt2, W2.T, b2)
    state = _gather_kernel(h3, u3)
    return (h3, atom, state)
